# baseline (speedup 1.0000x reference)
import functools

import jax
import jax.numpy as jnp
from jax import lax
from jax.experimental import pallas as pl
from jax.experimental.pallas import tpu as pltpu
from jax.experimental.pallas import tpu_sc as plsc

_B, _S, _K, _H = 16, 2048, 16, 128
_NC, _NS, _L = 2, 16, 16
_NW = _NC * _NS
_ROWS = _B * _S
_SS = _ROWS // _NW
_CH = 128
_NCHUNK = _SS // _CH
_G = 4
_KG = _K // _G
_ZBASE = _ROWS
_Z = 8 * _S


def _gather_body(tab_hbm, ni_hbm, out_hbm, ni0_v, ni1_v, mask_v,
                 acc0_v, acc1_v, acc2_v, acc3_v, outst_v, sem_g, sem_o):
    accs = [acc0_v, acc1_v, acc2_v, acc3_v]
    nis = [ni0_v, ni1_v]
    wid = lax.axis_index("s") * _NC + lax.axis_index("c")
    row0 = wid * _SS
    b = row0 // _S
    boff = b * _S
    scol0 = row0 % _S
    zbase = _ZBASE + (wid % 8) * _S

    def load_indices(c, dst):
        scol = scol0 + c * _CH
        pltpu.sync_copy(ni_hbm.at[pl.ds(b * 2 * _K, _K), pl.ds(scol, _CH)],
                        dst)
        pltpu.sync_copy(
            ni_hbm.at[pl.ds(b * 2 * _K + _K, _K), pl.ds(scol, _CH)], mask_v)

        def vec_body(j, carry):
            k = j // (_CH // _L)
            off = (j % (_CH // _L)) * _L
            sl = pl.ds(off, _L)
            ni = dst[k, sl]
            v = mask_v[k, sl] * jnp.minimum(ni, 1)
            zslot = ((wid % 8) * _S + scol + off + k * _CH
                     + lax.iota(jnp.int32, _L)) & (_Z - 1)
            dst[k, sl] = v * (ni - 1 + boff) + (1 - v) * (_ZBASE + zslot)
            return carry

        lax.fori_loop(0, _K * _CH // _L, vec_body, 0)

    load_indices(0, nis[0])
    out_desc = None
    for c in range(_NCHUNK):
        cur = nis[c % 2]
        inits = [
            pltpu.async_copy(tab_hbm.at[cur.at[g * _KG]], accs[g], sem_g)
            for g in range(_G)
        ]
        for d in inits:
            d.wait()
        adds = [
            pltpu.async_copy(tab_hbm.at[cur.at[g * _KG + j]], accs[g],
                             sem_g, add=True)
            for g in range(_G) for j in range(1, _KG)
        ]
        if c + 1 < _NCHUNK:
            load_indices(c + 1, nis[(c + 1) % 2])
        for d in adds:
            d.wait()
        if out_desc is not None:
            out_desc.wait()

        def comb_body(r, carry):
            for v in range(_H // _L):
                sl = pl.ds(v * _L, _L)
                outst_v[r, sl] = ((accs[0][r, sl] + accs[1][r, sl])
                                  + (accs[2][r, sl] + accs[3][r, sl]))
            return carry

        lax.fori_loop(0, _CH, comb_body, 0)
        out_desc = pltpu.async_copy(
            outst_v, out_hbm.at[pl.ds(row0 + c * _CH, _CH), :], sem_o)
    out_desc.wait()


_gather_call = functools.partial(
    pl.kernel,
    out_type=jax.ShapeDtypeStruct((_ROWS, _H), jnp.float32),
    mesh=plsc.VectorSubcoreMesh(core_axis_name="c", subcore_axis_name="s"),
    scratch_types=[
        pltpu.VMEM((_K, _CH), jnp.int32),
        pltpu.VMEM((_K, _CH), jnp.int32),
        pltpu.VMEM((_K, _CH), jnp.int32),
        pltpu.VMEM((_CH, _H), jnp.float32),
        pltpu.VMEM((_CH, _H), jnp.float32),
        pltpu.VMEM((_CH, _H), jnp.float32),
        pltpu.VMEM((_CH, _H), jnp.float32),
        pltpu.VMEM((_CH, _H), jnp.float32),
        pltpu.SemaphoreType.DMA,
        pltpu.SemaphoreType.DMA,
    ],
)(_gather_body)


_PB = 512
_NPB = (_ROWS + _Z) // _PB
_NHB = _ROWS // _PB


def _prep_body(h_ref, w_ref, o_ref):
    i = pl.program_id(0)

    @pl.when(i < _NHB)
    def _():
        o_ref[...] = 0.0625 * lax.dot_general(
            h_ref[...], w_ref[...], (((1,), (1,)), ((), ())),
            preferred_element_type=jnp.float32,
            precision=lax.Precision.HIGHEST)

    @pl.when(i >= _NHB)
    def _():
        o_ref[...] = jnp.zeros_like(o_ref)


_prep_call = pl.pallas_call(
    _prep_body,
    grid=(_NPB,),
    in_specs=[
        pl.BlockSpec((_PB, _H), lambda i: (jnp.minimum(i, _NHB - 1), 0)),
        pl.BlockSpec((_H, _H), lambda i: (0, 0)),
    ],
    out_specs=pl.BlockSpec((_PB, _H), lambda i: (i, 0)),
    out_shape=jax.ShapeDtypeStruct((_ROWS + _Z, _H), jnp.float32),
)


def kernel(x, h, g, neighbor_index, neighbor_mask, Wh, Wn, U, u_w, u_b,
           V_w, V_b):
    tab = _prep_call(h.reshape(_ROWS, _H), Wn)
    nm = jnp.concatenate([neighbor_index, neighbor_mask], axis=2)
    nmT = nm.transpose(0, 2, 1).reshape(_B * 2 * _K, _S)
    out = _gather_call(tab, nmT)
    return out.reshape(_B, _S, _H)


# speedup vs baseline: 1.0249x; 1.0249x over previous
import functools

import jax
import jax.numpy as jnp
from jax import lax
from jax.experimental import pallas as pl
from jax.experimental.pallas import tpu as pltpu
from jax.experimental.pallas import tpu_sc as plsc

_B, _S, _K, _H = 16, 2048, 16, 128
_NC, _NS, _L = 2, 16, 16
_NW = _NC * _NS
_ROWS = _B * _S
_SS = _ROWS // _NW
_CH = 128
_NCHUNK = _SS // _CH
_G = 4
_KG = _K // _G
_ZBASE = _ROWS
_Z = 8 * _S


def _gather_body(tab_hbm, ni_hbm, mask_hbm, out_hbm, ni0_v, ni1_v, mask_v,
                 acc0_v, acc1_v, acc2_v, acc3_v, outst_v, sem_g, sem_o):
    accs = [acc0_v, acc1_v, acc2_v, acc3_v]
    nis = [ni0_v, ni1_v]
    wid = lax.axis_index("s") * _NC + lax.axis_index("c")
    row0 = wid * _SS
    b = row0 // _S
    boff = b * _S
    scol0 = row0 % _S
    zbase = _ZBASE + (wid % 8) * _S

    def load_indices(c, dst):
        scol = scol0 + c * _CH
        pltpu.sync_copy(ni_hbm.at[pl.ds(b * _K, _K), pl.ds(scol, _CH)], dst)
        pltpu.sync_copy(mask_hbm.at[pl.ds(b * _K, _K), pl.ds(scol, _CH)],
                        mask_v)

        def vec_body(j, carry):
            k = j // (_CH // _L)
            off = (j % (_CH // _L)) * _L
            sl = pl.ds(off, _L)
            ni = dst[k, sl]
            v = mask_v[k, sl] * jnp.minimum(ni, 1)
            zslot = ((wid % 8) * _S + scol + off + k * _CH
                     + lax.iota(jnp.int32, _L)) & (_Z - 1)
            dst[k, sl] = v * (ni - 1 + boff) + (1 - v) * (_ZBASE + zslot)
            return carry

        lax.fori_loop(0, _K * _CH // _L, vec_body, 0)

    load_indices(0, nis[0])

    def zero_body(r, carry):
        z = jnp.zeros((_L,), jnp.float32)
        for v in range(_H // _L):
            sl = pl.ds(v * _L, _L)
            for g in range(_G):
                accs[g][r, sl] = z
        return carry

    lax.fori_loop(0, _CH, zero_body, 0)

    out_desc = None
    for c in range(_NCHUNK):
        cur = nis[c % 2]
        adds = [
            pltpu.async_copy(tab_hbm.at[cur.at[g * _KG + j]], accs[g],
                             sem_g, add=True)
            for g in range(_G) for j in range(_KG)
        ]
        if c + 1 < _NCHUNK:
            load_indices(c + 1, nis[(c + 1) % 2])
        for d in adds:
            d.wait()
        if out_desc is not None:
            out_desc.wait()

        def comb_body(r, carry):
            z = jnp.zeros((_L,), jnp.float32)
            for v in range(_H // _L):
                sl = pl.ds(v * _L, _L)
                t0 = accs[0][r, sl] + accs[1][r, sl]
                t1 = accs[2][r, sl] + accs[3][r, sl]
                outst_v[r, sl] = t0 + t1
                for g in range(_G):
                    accs[g][r, sl] = z
            return carry

        lax.fori_loop(0, _CH, comb_body, 0)
        out_desc = pltpu.async_copy(
            outst_v, out_hbm.at[pl.ds(row0 + c * _CH, _CH), :], sem_o)
    out_desc.wait()


_gather_call = functools.partial(
    pl.kernel,
    out_type=jax.ShapeDtypeStruct((_ROWS, _H), jnp.float32),
    mesh=plsc.VectorSubcoreMesh(core_axis_name="c", subcore_axis_name="s"),
    scratch_types=[
        pltpu.VMEM((_K, _CH), jnp.int32),
        pltpu.VMEM((_K, _CH), jnp.int32),
        pltpu.VMEM((_K, _CH), jnp.int32),
        pltpu.VMEM((_CH, _H), jnp.float32),
        pltpu.VMEM((_CH, _H), jnp.float32),
        pltpu.VMEM((_CH, _H), jnp.float32),
        pltpu.VMEM((_CH, _H), jnp.float32),
        pltpu.VMEM((_CH, _H), jnp.float32),
        pltpu.SemaphoreType.DMA,
        pltpu.SemaphoreType.DMA,
    ],
)(_gather_body)


_PB = 512
_NPB = (_ROWS + _Z) // _PB
_NHB = _ROWS // _PB


def _prep_body(h_ref, w_ref, o_ref):
    i = pl.program_id(0)

    @pl.when(i < _NHB)
    def _():
        o_ref[...] = 0.0625 * lax.dot_general(
            h_ref[...], w_ref[...], (((1,), (1,)), ((), ())),
            preferred_element_type=jnp.float32,
            precision=lax.Precision.HIGHEST)

    @pl.when(i >= _NHB)
    def _():
        o_ref[...] = jnp.zeros_like(o_ref)


_prep_call = pl.pallas_call(
    _prep_body,
    grid=(_NPB,),
    in_specs=[
        pl.BlockSpec((_PB, _H), lambda i: (jnp.minimum(i, _NHB - 1), 0)),
        pl.BlockSpec((_H, _H), lambda i: (0, 0)),
    ],
    out_specs=pl.BlockSpec((_PB, _H), lambda i: (i, 0)),
    out_shape=jax.ShapeDtypeStruct((_ROWS + _Z, _H), jnp.float32),
)


def kernel(x, h, g, neighbor_index, neighbor_mask, Wh, Wn, U, u_w, u_b,
           V_w, V_b):
    tab = _prep_call(h.reshape(_ROWS, _H), Wn)
    ni = neighbor_index.transpose(0, 2, 1).reshape(_B * _K, _S)
    mask = neighbor_mask.transpose(0, 2, 1).reshape(_B * _K, _S)
    out = _gather_call(tab, ni, mask)
    return out.reshape(_B, _S, _H)


# speedup vs baseline: 1.0398x; 1.0145x over previous
import functools

import jax
import jax.numpy as jnp
from jax import lax
from jax.experimental import pallas as pl
from jax.experimental.pallas import tpu as pltpu
from jax.experimental.pallas import tpu_sc as plsc

_B, _S, _K, _H = 16, 2048, 16, 128
_NC, _NS, _L = 2, 16, 16
_NW = _NC * _NS
_ROWS = _B * _S
_SS = _ROWS // _NW
_CH = 128
_NCHUNK = _SS // _CH
_G = 4
_KG = _K // _G
_ZBASE = _ROWS
_Z = 8 * _S


def _gather_body(tab_hbm, ni_hbm, mask_hbm, out_hbm, ni0_v, ni1_v, mask_v,
                 acc0_v, acc1_v, acc2_v, acc3_v, outst_v, sem_g, sem_o):
    accs = [acc0_v, acc1_v, acc2_v, acc3_v]
    nis = [ni0_v, ni1_v]
    wid = lax.axis_index("s") * _NC + lax.axis_index("c")
    row0 = wid * _SS
    b = row0 // _S
    boff = b * _S
    scol0 = row0 % _S
    zbase = _ZBASE + (wid % 8) * _S

    def load_indices(c, dst):
        scol = scol0 + c * _CH
        pltpu.sync_copy(ni_hbm.at[pl.ds(b * _K, _K), pl.ds(scol, _CH)], dst)
        pltpu.sync_copy(mask_hbm.at[pl.ds(b * _K, _K), pl.ds(scol, _CH)],
                        mask_v)

        def vec_body(j, carry):
            k = j // (_CH // _L)
            off = (j % (_CH // _L)) * _L
            sl = pl.ds(off, _L)
            ni = dst[k, sl]
            v = mask_v[k, sl] * jnp.minimum(ni, 1)
            zslot = ((wid % 8) * _S + scol + off + k * _CH
                     + lax.iota(jnp.int32, _L)) & (_Z - 1)
            dst[k, sl] = v * (ni - 1 + boff) + (1 - v) * (_ZBASE + zslot)
            return carry

        lax.fori_loop(0, _K * _CH // _L, vec_body, 0)

    load_indices(0, nis[0])
    out_desc = None
    for c in range(_NCHUNK):
        cur = nis[c % 2]
        inits = [
            pltpu.async_copy(tab_hbm.at[cur.at[g * _KG]], accs[g], sem_g)
            for g in range(_G)
        ]
        for d in inits:
            d.wait()
        adds = [
            pltpu.async_copy(tab_hbm.at[cur.at[g * _KG + j]], accs[g],
                             sem_g, add=True)
            for g in range(_G) for j in range(1, _KG)
        ]
        if c + 1 < _NCHUNK:
            load_indices(c + 1, nis[(c + 1) % 2])
        for d in adds:
            d.wait()
        if out_desc is not None:
            out_desc.wait()

        def comb_body(r, carry):
            for v in range(_H // _L):
                sl = pl.ds(v * _L, _L)
                outst_v[r, sl] = ((accs[0][r, sl] + accs[1][r, sl])
                                  + (accs[2][r, sl] + accs[3][r, sl]))
            return carry

        lax.fori_loop(0, _CH, comb_body, 0)
        out_desc = pltpu.async_copy(
            outst_v, out_hbm.at[pl.ds(row0 + c * _CH, _CH), :], sem_o)
    out_desc.wait()


_gather_call = functools.partial(
    pl.kernel,
    out_type=jax.ShapeDtypeStruct((_ROWS, _H), jnp.float32),
    mesh=plsc.VectorSubcoreMesh(core_axis_name="c", subcore_axis_name="s"),
    scratch_types=[
        pltpu.VMEM((_K, _CH), jnp.int32),
        pltpu.VMEM((_K, _CH), jnp.int32),
        pltpu.VMEM((_K, _CH), jnp.int32),
        pltpu.VMEM((_CH, _H), jnp.float32),
        pltpu.VMEM((_CH, _H), jnp.float32),
        pltpu.VMEM((_CH, _H), jnp.float32),
        pltpu.VMEM((_CH, _H), jnp.float32),
        pltpu.VMEM((_CH, _H), jnp.float32),
        pltpu.SemaphoreType.DMA,
        pltpu.SemaphoreType.DMA,
    ],
)(_gather_body)


_PB = 512
_NPB = (_ROWS + _Z) // _PB
_NHB = _ROWS // _PB


def _prep_body(h_ref, w_ref, o_ref):
    i = pl.program_id(0)

    @pl.when(i < _NHB)
    def _():
        o_ref[...] = 0.0625 * lax.dot_general(
            h_ref[...], w_ref[...], (((1,), (1,)), ((), ())),
            preferred_element_type=jnp.float32,
            precision=lax.Precision.HIGHEST)

    @pl.when(i >= _NHB)
    def _():
        o_ref[...] = jnp.zeros_like(o_ref)


_prep_call = pl.pallas_call(
    _prep_body,
    grid=(_NPB,),
    in_specs=[
        pl.BlockSpec((_PB, _H), lambda i: (jnp.minimum(i, _NHB - 1), 0)),
        pl.BlockSpec((_H, _H), lambda i: (0, 0)),
    ],
    out_specs=pl.BlockSpec((_PB, _H), lambda i: (i, 0)),
    out_shape=jax.ShapeDtypeStruct((_ROWS + _Z, _H), jnp.float32),
)


def kernel(x, h, g, neighbor_index, neighbor_mask, Wh, Wn, U, u_w, u_b,
           V_w, V_b):
    tab = _prep_call(h.reshape(_ROWS, _H), Wn)
    ni = neighbor_index.transpose(0, 2, 1).reshape(_B * _K, _S)
    mask = neighbor_mask.transpose(0, 2, 1).reshape(_B * _K, _S)
    out = _gather_call(tab, ni, mask)
    return out.reshape(_B, _S, _H)
